# Initial kernel scaffold; baseline (speedup 1.0000x reference)
import jax
import jax.numpy as jnp
from jax.experimental import pallas as pl


def kernel(x, x_lengths, edge_list, W_msg, b_msg, W_gru, U_gru, b_gru):
    raise NotImplementedError("write your pallas kernel here")


# speedup vs baseline: 3.4868x; 3.4868x over previous
import functools

import jax
import jax.numpy as jnp
from jax import lax
from jax.experimental import pallas as pl
from jax.experimental.pallas import tpu as pltpu
from jax.experimental.pallas import tpu_sc as plsc

N_NODES = 10000
D = 128
N_EDGE_TYPES = 3
E_PER_TYPE = 213334
N_LAYERS = 3

NC = 2
NS = 16
N_TILES = NC * NS
CHUNK = 128
CHUNKS_PER_TILE = -(-E_PER_TYPE // (N_TILES * CHUNK))
E_PAD = N_TILES * CHUNK * CHUNKS_PER_TILE
ACC_ROWS = 10240
DUMMY_DST = N_NODES
ROWS_PER_SUBCORE_ZERO = ACC_ROWS // NS
ROWS_PER_SUBCORE_OUT = N_NODES // NS

BR = 1000
NBLK = N_NODES // BR


def _msg_body(h_ref, w_ref, b_ref, out_ref):
    out_ref[0] = (
        jnp.dot(h_ref[...], w_ref[0], preferred_element_type=jnp.float32)
        + b_ref[0]
    )


def _msg_matmul(h, Wm, bm):
    return pl.pallas_call(
        _msg_body,
        grid=(NBLK, N_EDGE_TYPES),
        in_specs=[
            pl.BlockSpec((BR, D), lambda i, t: (i, 0)),
            pl.BlockSpec((1, D, D), lambda i, t: (t, 0, 0)),
            pl.BlockSpec((1, 1, D), lambda i, t: (t, 0, 0)),
        ],
        out_specs=pl.BlockSpec((1, BR, D), lambda i, t: (t, i, 0)),
        out_shape=jax.ShapeDtypeStruct((N_EDGE_TYPES, N_NODES, D), jnp.float32),
    )(h, Wm, bm.reshape(N_EDGE_TYPES, 1, D))


def _gru_body(p_ref, h_ref, wg_ref, ug_ref, bg_ref, out_ref):
    a = p_ref[0] + p_ref[1]
    h = h_ref[...]
    dot = functools.partial(jnp.dot, preferred_element_type=jnp.float32)
    z = jax.nn.sigmoid(dot(a, wg_ref[0]) + dot(h, ug_ref[0]) + bg_ref[0])
    r = jax.nn.sigmoid(dot(a, wg_ref[1]) + dot(h, ug_ref[1]) + bg_ref[1])
    ht = jnp.tanh(dot(a, wg_ref[2]) + dot(r * h, ug_ref[2]) + bg_ref[2])
    out_ref[...] = (1.0 - z) * h + z * ht


def _gru_update(parts, h, Wg, Ug, bg):
    return pl.pallas_call(
        _gru_body,
        grid=(NBLK,),
        in_specs=[
            pl.BlockSpec((NC, BR, D), lambda i: (0, i, 0)),
            pl.BlockSpec((BR, D), lambda i: (i, 0)),
            pl.BlockSpec((3, D, D), lambda i: (0, 0, 0)),
            pl.BlockSpec((3, D, D), lambda i: (0, 0, 0)),
            pl.BlockSpec((3, 1, D), lambda i: (0, 0, 0)),
        ],
        out_specs=pl.BlockSpec((BR, D), lambda i: (i, 0)),
        out_shape=jax.ShapeDtypeStruct((N_NODES, D), jnp.float32),
    )(parts, h, Wg, Ug, bg.reshape(3, 1, D))


def _edge_pass_body(xw_hbm, src_hbm, dst_hbm, out_hbm,
                    acc, src_v, dst_v, rows_v):
    cid = lax.axis_index("c")
    sid = lax.axis_index("s")
    tile = cid * NS + sid

    @pl.loop(0, CHUNK)
    def _(i):
        @pl.loop(0, D, step=16)
        def _(j):
            rows_v[i, pl.ds(j, 16)] = jnp.zeros((16,), jnp.float32)

    zbase = sid * ROWS_PER_SUBCORE_ZERO
    @pl.loop(0, ROWS_PER_SUBCORE_ZERO // CHUNK)
    def _(k):
        pltpu.sync_copy(rows_v, acc.at[pl.ds(zbase + k * CHUNK, CHUNK)])

    plsc.subcore_barrier()

    for t in range(N_EDGE_TYPES):
        @pl.loop(0, CHUNKS_PER_TILE)
        def _(c):
            start = pl.multiple_of(
                t * E_PAD + (tile * CHUNKS_PER_TILE + c) * CHUNK, CHUNK)
            pltpu.sync_copy(src_hbm.at[pl.ds(start, CHUNK)], src_v)
            pltpu.sync_copy(dst_hbm.at[pl.ds(start, CHUNK)], dst_v)
            pltpu.sync_copy(xw_hbm.at[t].at[src_v], rows_v)
            pltpu.sync_copy(rows_v, acc.at[dst_v], add=True)

    plsc.subcore_barrier()

    obase = sid * ROWS_PER_SUBCORE_ZERO
    pltpu.sync_copy(acc.at[pl.ds(obase, ROWS_PER_SUBCORE_ZERO)],
                    out_hbm.at[cid].at[pl.ds(obase, ROWS_PER_SUBCORE_ZERO)])


def _edge_pass(xw, src, dst):
    mesh = plsc.VectorSubcoreMesh(core_axis_name="c", subcore_axis_name="s")
    k = pl.kernel(
        _edge_pass_body,
        out_type=jax.ShapeDtypeStruct((NC, ACC_ROWS, D), jnp.float32),
        mesh=mesh,
        scratch_types=[
            pltpu.VMEM_SHARED((ACC_ROWS, D), jnp.float32),
            pltpu.VMEM((CHUNK,), jnp.int32),
            pltpu.VMEM((CHUNK,), jnp.int32),
            pltpu.VMEM((CHUNK, D), jnp.float32),
        ],
    )
    return k(xw, src, dst)


def kernel(x, x_lengths, edge_list, W_msg, b_msg, W_gru, U_gru, b_gru):
    del x_lengths
    src = edge_list[:, 0, :]
    dst = edge_list[:, 1, :]
    pad = E_PAD - E_PER_TYPE
    src = jnp.pad(src, ((0, 0), (0, pad))).reshape(-1)
    dst = jnp.pad(dst, ((0, 0), (0, pad)),
                  constant_values=DUMMY_DST).reshape(-1)

    h = x
    for l in range(N_LAYERS):
        xw = _msg_matmul(h, W_msg[l], b_msg[l])
        parts = _edge_pass(xw, src, dst)
        h = _gru_update(parts, h, W_gru[l], U_gru[l], b_gru[l])
    return h
